# baseline (speedup 1.0000x reference)
import functools

import jax
import jax.numpy as jnp
from jax import lax
from jax.experimental import pallas as pl
from jax.experimental.pallas import tpu as pltpu
from jax.experimental.pallas import tpu_sc as plsc

N = 10000
D = 128
HD = D // 2
ALPHA = 0.15
SCALING = 1.8

NC = 2
NS = 16
NW = NC * NS
L = 16
CH = 128
NBUF = 4
ACC_ROWS = 10240
RPT = ACC_ROWS // NS

f32 = jnp.float32
i32 = jnp.int32

_sc_params = pltpu.CompilerParams(use_tc_tiling_on_sc=False)


def _pad_edges(e_total):
  per = NW * CH * NBUF
  return ((e_total + per - 1) // per) * per


def _sc_mesh():
  return plsc.VectorSubcoreMesh(
      core_axis_name="c", subcore_axis_name="s", num_cores=NC, num_subcores=NS)


def _make_sc_degree(e_pad):
  ncha = e_pad // (NS * CH)
  nch2 = ncha // NC

  @functools.partial(
      pl.kernel,
      out_type=jax.ShapeDtypeStruct((NC, ACC_ROWS, L), f32),
      mesh=_sc_mesh(),
      scratch_types=[
          pltpu.VMEM_SHARED((ACC_ROWS, L), f32),
          pltpu.VMEM((CH, L), f32),
          pltpu.VMEM((nch2, CH), i32),
          pltpu.SemaphoreType.DMA,
      ],
      compiler_params=_sc_params,
  )
  def deg_kernel(dst2_hbm, zero_hbm, ones_hbm, out_hbm,
                 acc, ones_v, didx, ssem):
    c = lax.axis_index("c")
    s = lax.axis_index("s")
    pltpu.sync_copy(ones_hbm, ones_v)
    pltpu.sync_copy(dst2_hbm.at[pl.ds(s * ncha + c * nch2, nch2)], didx)
    pltpu.sync_copy(zero_hbm, acc.at[pl.ds(s * RPT, RPT)])
    plsc.subcore_barrier()

    def grp(t, carry):
      for b in range(NBUF):
        pltpu.async_copy(ones_v, acc.at[didx.at[t * NBUF + b]], ssem,
                         add=True)
      for b in range(NBUF):
        pltpu.make_async_copy(ones_v, acc.at[didx.at[t * NBUF + b]],
                              ssem).wait()
      return carry

    lax.fori_loop(0, nch2 // NBUF, grp, 0)
    plsc.subcore_barrier()
    pltpu.sync_copy(acc.at[pl.ds(s * RPT, RPT)],
                    out_hbm.at[c, pl.ds(s * RPT, RPT)])

  return deg_kernel


def _make_sc_prop(e_pad):
  ncha = e_pad // (NS * CH)
  nouter = ncha // NBUF

  @functools.partial(
      pl.kernel,
      out_type=jax.ShapeDtypeStruct((NC, ACC_ROWS, HD), f32),
      mesh=_sc_mesh(),
      scratch_types=[
          pltpu.VMEM_SHARED((ACC_ROWS, HD), f32),
          pltpu.VMEM((NBUF, CH, HD), f32),
          pltpu.VMEM((ncha, CH), i32),
          pltpu.VMEM((ncha, CH), i32),
      ] + [pltpu.SemaphoreType.DMA] * (2 * NBUF),
      compiler_params=_sc_params,
  )
  def prop_kernel(g_hbm, src2_hbm, dst2_hbm, zero_hbm, out_hbm,
                  acc, rows, sidx, didx, *sems):
    gsem = sems[:NBUF]
    ssem = sems[NBUF:]
    c = lax.axis_index("c")
    s = lax.axis_index("s")
    pltpu.sync_copy(src2_hbm.at[pl.ds((c * NS + s) * ncha, ncha)], sidx)
    pltpu.sync_copy(dst2_hbm.at[pl.ds(s * ncha, ncha)], didx)
    pltpu.sync_copy(zero_hbm, acc.at[pl.ds(s * RPT, RPT)])
    plsc.subcore_barrier()

    for b in range(NBUF):
      pltpu.async_copy(g_hbm.at[sidx.at[b]], rows.at[b], gsem[b])

    def outer(t, carry):
      j0 = t * NBUF
      for b in range(NBUF):
        pltpu.make_async_copy(g_hbm.at[sidx.at[j0 + b]], rows.at[b],
                              gsem[b]).wait()
        pltpu.async_copy(rows.at[b], acc.at[didx.at[j0 + b]], ssem[b],
                         add=True)
      for b in range(NBUF):
        pltpu.make_async_copy(rows.at[b], acc.at[didx.at[j0 + b]],
                              ssem[b]).wait()

        @pl.when(j0 + b + NBUF < ncha)
        def _():
          pltpu.async_copy(g_hbm.at[sidx.at[j0 + b + NBUF]], rows.at[b],
                           gsem[b])

      return carry

    lax.fori_loop(0, nouter, outer, 0)
    plsc.subcore_barrier()
    pltpu.sync_copy(acc.at[pl.ds(s * RPT, RPT)],
                    out_hbm.at[c, pl.ds(s * RPT, RPT)])

  return prop_kernel


R = 2000
GRID = N // R


def _dinv(d_ref):
  dall = d_ref[...]
  d = jnp.sum(dall, axis=0)[:, 0:1]
  return jnp.where(d > 0.0, 1.0 / jnp.sqrt(d), 0.0)


def _agg(p_ref):
  p = p_ref[...]
  return jnp.concatenate([p[0], p[1]], axis=1)


def _l2n(h, scale):
  n = jnp.sqrt(jnp.sum(h * h, axis=1, keepdims=True))
  return h / jnp.maximum(n, 1e-12) * scale


def _matmul_stage(x, w_ref, b_ref):
  h = jnp.dot(x, w_ref[...], preferred_element_type=f32) + b_ref[...]
  return _l2n(h, SCALING)


def _store_g(g_ref, gh):
  g_ref[0] = gh[:, :HD]
  g_ref[1] = gh[:, HD:]


def _tc_stage0_body(x_ref, w_ref, b_ref, d_ref, h0_ref, g_ref):
  h0 = _matmul_stage(x_ref[...], w_ref, b_ref)
  dinv = _dinv(d_ref)
  h0_ref[...] = h0
  _store_g(g_ref, h0 * dinv)


def _tc_mid_body(p_ref, d_ref, h0_ref, g_ref):
  dinv = _dinv(d_ref)
  h1 = (1.0 - ALPHA) * dinv * _agg(p_ref) + ALPHA * h0_ref[...]
  _store_g(g_ref, h1 * dinv)


def _tc_end_body(p_ref, d_ref, h0_ref, w_ref, b_ref, h0n_ref, g_ref, *, act):
  dinv = _dinv(d_ref)
  h2 = (1.0 - ALPHA) * dinv * _agg(p_ref) + ALPHA * h0_ref[...]
  if act == "relu":
    xn = jnp.maximum(h2, 0.0)
  else:
    xn = _l2n(h2, 1.5)
  h0n = _matmul_stage(xn, w_ref, b_ref)
  h0n_ref[...] = h0n
  _store_g(g_ref, h0n * dinv)


def _tc_final_body(p_ref, d_ref, h0_ref, out_ref):
  dinv = _dinv(d_ref)
  out_ref[...] = (1.0 - ALPHA) * dinv * _agg(p_ref) + ALPHA * h0_ref[...]


_rows_spec = pl.BlockSpec((R, D), lambda i: (i, 0))
_p_spec = pl.BlockSpec((NC, R, HD), lambda i: (0, i, 0))
_g_spec = pl.BlockSpec((NC, R, HD), lambda i: (0, i, 0))
_d_spec = pl.BlockSpec((NC, R, L), lambda i: (0, i, 0))
_w_spec = pl.BlockSpec((D, D), lambda i: (0, 0))
_b_spec = pl.BlockSpec((1, D), lambda i: (0, 0))
_nd = jax.ShapeDtypeStruct((N, D), f32)
_g_shape = jax.ShapeDtypeStruct((NC, N, HD), f32)


def _tc_stage0(x, w, b, degp):
  return pl.pallas_call(
      _tc_stage0_body,
      grid=(GRID,),
      in_specs=[_rows_spec, _w_spec, _b_spec, _d_spec],
      out_specs=[_rows_spec, _g_spec],
      out_shape=[_nd, _g_shape],
  )(x, w, b, degp)


def _tc_mid(p, degp, h0):
  return pl.pallas_call(
      _tc_mid_body,
      grid=(GRID,),
      in_specs=[_p_spec, _d_spec, _rows_spec],
      out_specs=[_g_spec],
      out_shape=[_g_shape],
  )(p, degp, h0)[0]


def _tc_end(p, degp, h0, w, b, act):
  return pl.pallas_call(
      functools.partial(_tc_end_body, act=act),
      grid=(GRID,),
      in_specs=[_p_spec, _d_spec, _rows_spec, _w_spec, _b_spec],
      out_specs=[_rows_spec, _g_spec],
      out_shape=[_nd, _g_shape],
  )(p, degp, h0, w, b)


def _tc_final(p, degp, h0):
  return pl.pallas_call(
      _tc_final_body,
      grid=(GRID,),
      in_specs=[_p_spec, _d_spec, _rows_spec],
      out_specs=[_rows_spec],
      out_shape=[_nd],
  )(p, degp, h0)[0]


@jax.jit
def kernel(x, edge_index, W0, b0, W1, b1, Wx, bx):
  e_total = edge_index.shape[1] + N
  e_pad = _pad_edges(e_total)
  npad = e_pad - e_total
  ncha = e_pad // (NS * CH)
  loop = jnp.arange(N, dtype=i32)
  src = jnp.concatenate([edge_index[0].astype(i32), loop,
                         jnp.zeros((npad,), i32)])
  dst = jnp.concatenate([edge_index[1].astype(i32), loop,
                         jnp.full((npad,), N, i32)])
  src2 = jnp.stack([src, src + N]).reshape(NC * NS * ncha, CH)
  dst2 = dst.reshape(NS * ncha, CH)

  zero_rows = jnp.zeros((RPT, HD), f32)
  zero_rows16 = jnp.zeros((RPT, L), f32)
  ones16 = jnp.ones((CH, L), f32)

  sc_deg = _make_sc_degree(e_pad)
  sc_prop = _make_sc_prop(e_pad)

  degp = sc_deg(dst2, zero_rows16, ones16)

  b0r = b0.reshape(1, D)
  b1r = b1.reshape(1, D)
  bxr = bx.reshape(1, D)

  def prop(g):
    return sc_prop(g.reshape(NC * N, HD), src2, dst2, zero_rows)

  h0, g = _tc_stage0(x, W0, b0r, degp)
  g = _tc_mid(prop(g), degp, h0)
  h0, g = _tc_end(prop(g), degp, h0, W1, b1r, "relu")
  g = _tc_mid(prop(g), degp, h0)
  h0, g = _tc_end(prop(g), degp, h0, Wx, bxr, "l2n")
  g = _tc_mid(prop(g), degp, h0)
  return _tc_final(prop(g), degp, h0)


# speedup vs baseline: 2.2736x; 2.2736x over previous
import functools

import jax
import jax.numpy as jnp
from jax import lax
from jax.experimental import pallas as pl
from jax.experimental.pallas import tpu as pltpu
from jax.experimental.pallas import tpu_sc as plsc

N = 10000
D = 128
HD = D // 2
ALPHA = 0.15
SCALING = 1.8

NC = 2
NS = 16
NW = NC * NS
L = 16
CH = 128
NBUF = 4
ACC_ROWS = 10240
RPT = ACC_ROWS // NS

f32 = jnp.float32
i32 = jnp.int32

_sc_params = pltpu.CompilerParams(use_tc_tiling_on_sc=False)


def _pad_edges(e_total):
  per = NW * CH * NBUF
  return ((e_total + per - 1) // per) * per


def _sc_mesh():
  return plsc.VectorSubcoreMesh(
      core_axis_name="c", subcore_axis_name="s", num_cores=NC, num_subcores=NS)


def _make_sc_degree(e_pad):
  ncha = e_pad // (NS * CH)
  nch2 = ncha // NC

  @functools.partial(
      pl.kernel,
      out_type=jax.ShapeDtypeStruct((NC, ACC_ROWS, L), f32),
      mesh=_sc_mesh(),
      scratch_types=[
          pltpu.VMEM_SHARED((ACC_ROWS, L), f32),
          pltpu.VMEM((CH, L), f32),
          pltpu.VMEM((nch2, CH), i32),
          pltpu.SemaphoreType.DMA,
      ],
      compiler_params=_sc_params,
  )
  def deg_kernel(dst2_hbm, zero_hbm, ones_hbm, out_hbm,
                 acc, ones_v, didx, ssem):
    c = lax.axis_index("c")
    s = lax.axis_index("s")
    pltpu.sync_copy(ones_hbm, ones_v)
    pltpu.sync_copy(dst2_hbm.at[pl.ds(s * ncha + c * nch2, nch2)], didx)
    pltpu.sync_copy(zero_hbm, acc.at[pl.ds(s * RPT, RPT)])
    plsc.subcore_barrier()

    def grp(t, carry):
      for b in range(NBUF):
        pltpu.async_copy(ones_v, acc.at[didx.at[t * NBUF + b]], ssem,
                         add=True)
      for b in range(NBUF):
        pltpu.make_async_copy(ones_v, acc.at[didx.at[t * NBUF + b]],
                              ssem).wait()
      return carry

    lax.fori_loop(0, nch2 // NBUF, grp, 0)
    plsc.subcore_barrier()
    pltpu.sync_copy(acc.at[pl.ds(s * RPT, RPT)],
                    out_hbm.at[c, pl.ds(s * RPT, RPT)])

  return deg_kernel


def _make_sc_prop(e_pad):
  ncha = e_pad // (NS * CH)
  ngrp = ncha // NBUF
  rpt_g = N // NS

  @functools.partial(
      pl.kernel,
      out_type=jax.ShapeDtypeStruct((NC, ACC_ROWS, HD), f32),
      mesh=_sc_mesh(),
      scratch_types=[
          pltpu.VMEM_SHARED((ACC_ROWS, HD), f32),
          pltpu.VMEM_SHARED((N, HD), f32),
          pltpu.VMEM((NBUF, CH, HD), f32),
          pltpu.VMEM((2 * NBUF, CH), i32),
          pltpu.VMEM((2 * NBUF, CH), i32),
      ] + [pltpu.SemaphoreType.DMA] * (2 + 2 * NBUF),
      compiler_params=_sc_params,
  )
  def prop_kernel(g_hbm, src2_hbm, dst2_hbm, zero_hbm, out_hbm,
                  acc, gsp, rows, sidx_b, didx_b, *sems):
    isem = sems[:2]
    gsem = sems[2:2 + NBUF]
    ssem = sems[2 + NBUF:]
    c = lax.axis_index("c")
    s = lax.axis_index("s")
    base = s * ncha
    pltpu.sync_copy(zero_hbm, acc.at[pl.ds(s * RPT, RPT)])
    pltpu.sync_copy(g_hbm.at[pl.ds(c * N + s * rpt_g, rpt_g)],
                    gsp.at[pl.ds(s * rpt_g, rpt_g)])
    pltpu.async_copy(src2_hbm.at[pl.ds(base, NBUF)],
                     sidx_b.at[pl.ds(0, NBUF)], isem[0])
    pltpu.async_copy(dst2_hbm.at[pl.ds(base, NBUF)],
                     didx_b.at[pl.ds(0, NBUF)], isem[0])
    plsc.subcore_barrier()

    def half(t, p):
      pltpu.make_async_copy(src2_hbm.at[pl.ds(base, NBUF)],
                            sidx_b.at[pl.ds(p * NBUF, NBUF)], isem[p]).wait()
      pltpu.make_async_copy(dst2_hbm.at[pl.ds(base, NBUF)],
                            didx_b.at[pl.ds(p * NBUF, NBUF)], isem[p]).wait()
      for b in range(NBUF):
        @pl.when(t > 0)
        def _():
          pltpu.make_async_copy(rows.at[b], acc.at[didx_b.at[p * NBUF + b]],
                                ssem[b]).wait()

      @pl.when(t + 1 < ngrp)
      def _():
        pltpu.async_copy(src2_hbm.at[pl.ds(base + (t + 1) * NBUF, NBUF)],
                         sidx_b.at[pl.ds((1 - p) * NBUF, NBUF)], isem[1 - p])
        pltpu.async_copy(dst2_hbm.at[pl.ds(base + (t + 1) * NBUF, NBUF)],
                         didx_b.at[pl.ds((1 - p) * NBUF, NBUF)], isem[1 - p])

      for b in range(NBUF):
        pltpu.async_copy(gsp.at[sidx_b.at[p * NBUF + b]], rows.at[b],
                         gsem[b])
      for b in range(NBUF):
        pltpu.make_async_copy(gsp.at[sidx_b.at[p * NBUF + b]], rows.at[b],
                              gsem[b]).wait()
        pltpu.async_copy(rows.at[b], acc.at[didx_b.at[p * NBUF + b]],
                         ssem[b], add=True)

    def pair(u, carry):
      half(2 * u, 0)
      half(2 * u + 1, 1)
      return carry

    lax.fori_loop(0, ngrp // 2, pair, 0)
    for b in range(NBUF):
      pltpu.make_async_copy(rows.at[b], acc.at[didx_b.at[b]],
                            ssem[b]).wait()
    plsc.subcore_barrier()
    pltpu.sync_copy(acc.at[pl.ds(s * RPT, RPT)],
                    out_hbm.at[c, pl.ds(s * RPT, RPT)])

  return prop_kernel


R = 2000
GRID = N // R


def _dinv(d_ref):
  dall = d_ref[...]
  d = jnp.sum(dall, axis=0)[:, 0:1]
  return jnp.where(d > 0.0, 1.0 / jnp.sqrt(d), 0.0)


def _agg(p_ref):
  p = p_ref[...]
  return jnp.concatenate([p[0], p[1]], axis=1)


def _l2n(h, scale):
  n = jnp.sqrt(jnp.sum(h * h, axis=1, keepdims=True))
  return h / jnp.maximum(n, 1e-12) * scale


def _matmul_stage(x, w_ref, b_ref):
  h = jnp.dot(x, w_ref[...], preferred_element_type=f32) + b_ref[...]
  return _l2n(h, SCALING)


def _store_g(g_ref, gh):
  g_ref[0] = gh[:, :HD]
  g_ref[1] = gh[:, HD:]


def _tc_stage0_body(x_ref, w_ref, b_ref, d_ref, h0_ref, g_ref):
  h0 = _matmul_stage(x_ref[...], w_ref, b_ref)
  dinv = _dinv(d_ref)
  h0_ref[...] = h0
  _store_g(g_ref, h0 * dinv)


def _tc_mid_body(p_ref, d_ref, h0_ref, g_ref):
  dinv = _dinv(d_ref)
  h1 = (1.0 - ALPHA) * dinv * _agg(p_ref) + ALPHA * h0_ref[...]
  _store_g(g_ref, h1 * dinv)


def _tc_end_body(p_ref, d_ref, h0_ref, w_ref, b_ref, h0n_ref, g_ref, *, act):
  dinv = _dinv(d_ref)
  h2 = (1.0 - ALPHA) * dinv * _agg(p_ref) + ALPHA * h0_ref[...]
  if act == "relu":
    xn = jnp.maximum(h2, 0.0)
  else:
    xn = _l2n(h2, 1.5)
  h0n = _matmul_stage(xn, w_ref, b_ref)
  h0n_ref[...] = h0n
  _store_g(g_ref, h0n * dinv)


def _tc_final_body(p_ref, d_ref, h0_ref, out_ref):
  dinv = _dinv(d_ref)
  out_ref[...] = (1.0 - ALPHA) * dinv * _agg(p_ref) + ALPHA * h0_ref[...]


_rows_spec = pl.BlockSpec((R, D), lambda i: (i, 0))
_p_spec = pl.BlockSpec((NC, R, HD), lambda i: (0, i, 0))
_g_spec = pl.BlockSpec((NC, R, HD), lambda i: (0, i, 0))
_d_spec = pl.BlockSpec((NC, R, L), lambda i: (0, i, 0))
_w_spec = pl.BlockSpec((D, D), lambda i: (0, 0))
_b_spec = pl.BlockSpec((1, D), lambda i: (0, 0))
_nd = jax.ShapeDtypeStruct((N, D), f32)
_g_shape = jax.ShapeDtypeStruct((NC, N, HD), f32)


def _tc_stage0(x, w, b, degp):
  return pl.pallas_call(
      _tc_stage0_body,
      grid=(GRID,),
      in_specs=[_rows_spec, _w_spec, _b_spec, _d_spec],
      out_specs=[_rows_spec, _g_spec],
      out_shape=[_nd, _g_shape],
  )(x, w, b, degp)


def _tc_mid(p, degp, h0):
  return pl.pallas_call(
      _tc_mid_body,
      grid=(GRID,),
      in_specs=[_p_spec, _d_spec, _rows_spec],
      out_specs=[_g_spec],
      out_shape=[_g_shape],
  )(p, degp, h0)[0]


def _tc_end(p, degp, h0, w, b, act):
  return pl.pallas_call(
      functools.partial(_tc_end_body, act=act),
      grid=(GRID,),
      in_specs=[_p_spec, _d_spec, _rows_spec, _w_spec, _b_spec],
      out_specs=[_rows_spec, _g_spec],
      out_shape=[_nd, _g_shape],
  )(p, degp, h0, w, b)


def _tc_final(p, degp, h0):
  return pl.pallas_call(
      _tc_final_body,
      grid=(GRID,),
      in_specs=[_p_spec, _d_spec, _rows_spec],
      out_specs=[_rows_spec],
      out_shape=[_nd],
  )(p, degp, h0)[0]


@jax.jit
def kernel(x, edge_index, W0, b0, W1, b1, Wx, bx):
  e_total = edge_index.shape[1] + N
  e_pad = _pad_edges(e_total)
  npad = e_pad - e_total
  ncha = e_pad // (NS * CH)
  loop = jnp.arange(N, dtype=i32)
  src = jnp.concatenate([edge_index[0].astype(i32), loop,
                         jnp.zeros((npad,), i32)])
  dst = jnp.concatenate([edge_index[1].astype(i32), loop,
                         jnp.full((npad,), N, i32)])
  src2 = src.reshape(NS * ncha, CH)
  dst2 = dst.reshape(NS * ncha, CH)

  zero_rows = jnp.zeros((RPT, HD), f32)
  zero_rows16 = jnp.zeros((RPT, L), f32)
  ones16 = jnp.ones((CH, L), f32)

  sc_deg = _make_sc_degree(e_pad)
  sc_prop = _make_sc_prop(e_pad)

  degp = sc_deg(dst2, zero_rows16, ones16)

  b0r = b0.reshape(1, D)
  b1r = b1.reshape(1, D)
  bxr = bx.reshape(1, D)

  def prop(g):
    return sc_prop(g.reshape(NC * N, HD), src2, dst2, zero_rows)

  h0, g = _tc_stage0(x, W0, b0r, degp)
  g = _tc_mid(prop(g), degp, h0)
  h0, g = _tc_end(prop(g), degp, h0, W1, b1r, "relu")
  g = _tc_mid(prop(g), degp, h0)
  h0, g = _tc_end(prop(g), degp, h0, Wx, bxr, "l2n")
  g = _tc_mid(prop(g), degp, h0)
  return _tc_final(prop(g), degp, h0)
